# baseline (speedup 1.0000x reference)
import functools

import jax
import jax.numpy as jnp
from jax import lax
from jax.experimental import pallas as pl
from jax.experimental.pallas import tpu as pltpu
from jax.experimental.pallas import tpu_sc as plsc

B = 50000
D = 256
V_PAD = 104
CHUNK = 128
NW = 32
LANES = 16
N_FULL = B // CHUNK
TAIL = B - N_FULL * CHUNK
BASE_CPW = N_FULL // NW
EXTRA = N_FULL - BASE_CPW * NW
MAX_CPW = BASE_CPW + 1
IDXBUF = MAX_CPW * CHUNK
NBUF = 3


def _gather_kernel(idx_hbm, table_hbm, out_hbm, repl_hbm,
                   idx_v, tab_v, rows0, rows1, rows2,
                   sg0, sg1, sg2, ss0, ss1, ss2):
    wid = lax.axis_index("s") * 2 + lax.axis_index("c")
    nc = BASE_CPW + jnp.where(wid < EXTRA, 1, 0)
    base_chunk = BASE_CPW * wid + jnp.minimum(wid, EXTRA)
    base_row = base_chunk * CHUNK

    bufs = (rows0, rows1, rows2)
    sem_g = (sg0, sg1, sg2)
    sem_s = (ss0, ss1, ss2)

    pltpu.sync_copy(table_hbm, tab_v)
    pltpu.sync_copy(tab_v, repl_hbm.at[pl.ds(wid * V_PAD, V_PAD)])

    pltpu.sync_copy(idx_hbm.at[pl.ds(base_row, BASE_CPW * CHUNK)],
                    idx_v.at[pl.ds(0, BASE_CPW * CHUNK)])

    @pl.when(wid < EXTRA)
    def _():
        pltpu.sync_copy(idx_hbm.at[pl.ds(base_row + BASE_CPW * CHUNK, CHUNK)],
                        idx_v.at[pl.ds(BASE_CPW * CHUNK, CHUNK)])

    @pl.when(wid == NW - 1)
    def _():
        pltpu.sync_copy(idx_hbm.at[pl.ds(N_FULL * CHUNK, TAIL)],
                        idx_v.at[pl.ds(BASE_CPW * CHUNK, TAIL)])

    shift = wid * V_PAD

    def remap(k, _):
        sl = pl.ds(k * LANES, LANES)
        idx_v[sl] = idx_v[sl] + shift
        return 0

    lax.fori_loop(0, IDXBUF // LANES, remap, 0)

    def gather(i):
        return pltpu.make_async_copy(
            repl_hbm.at[idx_v.at[pl.ds(i * CHUNK, CHUNK)]],
            bufs[i % NBUF], sem_g[i % NBUF])

    def scatter(i):
        return pltpu.make_async_copy(
            bufs[i % NBUF], out_hbm.at[pl.ds(base_row + i * CHUNK, CHUNK)],
            sem_s[i % NBUF])

    gather(0).start()
    gather(1).start()
    for i in range(MAX_CPW):
        if i + 2 < MAX_CPW:
            @pl.when(i + 2 < nc)
            def _(i=i):
                if i >= 1:
                    scatter(i - 1).wait()
                gather(i + 2).start()

        @pl.when(i < nc)
        def _(i=i):
            gather(i).wait()
            scatter(i).start()

    scatter(0).wait()
    scatter(1).wait()
    scatter(2).wait()

    @pl.when(wid == NW - 1)
    def _():
        pltpu.async_copy(
            repl_hbm.at[idx_v.at[pl.ds(BASE_CPW * CHUNK, TAIL)]],
            rows0.at[pl.ds(0, TAIL)], sg0).wait()
        pltpu.sync_copy(rows0.at[pl.ds(0, TAIL)],
                        out_hbm.at[pl.ds(N_FULL * CHUNK, TAIL)])


@jax.jit
def _run(atomic_numbers, table_p):
    mesh = plsc.VectorSubcoreMesh(core_axis_name="c", subcore_axis_name="s")
    f = functools.partial(
        pl.kernel, mesh=mesh,
        out_type=(jax.ShapeDtypeStruct((B, D), jnp.float32),
                  jax.ShapeDtypeStruct((NW * V_PAD, D), jnp.float32)),
        scratch_types=[
            pltpu.VMEM((IDXBUF,), jnp.int32),
            pltpu.VMEM((V_PAD, D), jnp.float32),
            pltpu.VMEM((CHUNK, D), jnp.float32),
            pltpu.VMEM((CHUNK, D), jnp.float32),
            pltpu.VMEM((CHUNK, D), jnp.float32),
            pltpu.SemaphoreType.DMA,
            pltpu.SemaphoreType.DMA,
            pltpu.SemaphoreType.DMA,
            pltpu.SemaphoreType.DMA,
            pltpu.SemaphoreType.DMA,
            pltpu.SemaphoreType.DMA,
        ],
    )(_gather_kernel)
    out, _ = f(atomic_numbers, table_p)
    return out


def kernel(atomic_numbers, table):
    table_p = jnp.zeros((V_PAD, D), table.dtype).at[:table.shape[0]].set(table)
    return _run(atomic_numbers.astype(jnp.int32), table_p)


# speedup vs baseline: 1.1067x; 1.1067x over previous
import functools

import jax
import jax.numpy as jnp
from jax import lax
from jax.experimental import pallas as pl
from jax.experimental.pallas import tpu as pltpu
from jax.experimental.pallas import tpu_sc as plsc

B = 50000
D = 256
V_PAD = 128
CHUNK = 128
NW = 32
LANES = 16
N_FULL = B // CHUNK
TAIL = B - N_FULL * CHUNK
BASE_CPW = N_FULL // NW
EXTRA = N_FULL - BASE_CPW * NW
MAX_CPW = BASE_CPW + 1
IDXBUF = MAX_CPW * CHUNK


NBUF = 3


def _gather_kernel(idx_hbm, table_hbm, out_hbm,
                   idx_v, rows0, rows1, rows2, sg0, sg1, sg2, ss0, ss1, ss2):
    wid = lax.axis_index("s") * 2 + lax.axis_index("c")
    nc = BASE_CPW + jnp.where(wid < EXTRA, 1, 0)
    base_chunk = BASE_CPW * wid + jnp.minimum(wid, EXTRA)
    base_row = base_chunk * CHUNK

    bufs = (rows0, rows1, rows2)
    sem_g = (sg0, sg1, sg2)
    sem_s = (ss0, ss1, ss2)

    pltpu.sync_copy(idx_hbm.at[pl.ds(base_row, BASE_CPW * CHUNK)],
                    idx_v.at[pl.ds(0, BASE_CPW * CHUNK)])

    @pl.when(wid < EXTRA)
    def _():
        pltpu.sync_copy(idx_hbm.at[pl.ds(base_row + BASE_CPW * CHUNK, CHUNK)],
                        idx_v.at[pl.ds(BASE_CPW * CHUNK, CHUNK)])

    @pl.when(wid == NW - 1)
    def _():
        pltpu.sync_copy(idx_hbm.at[pl.ds(N_FULL * CHUNK, TAIL)],
                        idx_v.at[pl.ds(BASE_CPW * CHUNK, TAIL)])

    def remap(k, _):
        sl = pl.ds(k * LANES, LANES)
        idx_v[sl] = idx_v[sl] * NW + wid
        return 0

    lax.fori_loop(0, IDXBUF // LANES, remap, 0)

    def gather(i):
        return pltpu.make_async_copy(
            table_hbm.at[idx_v.at[pl.ds(i * CHUNK, CHUNK)]],
            bufs[i % NBUF], sem_g[i % NBUF])

    def scatter(i):
        return pltpu.make_async_copy(
            bufs[i % NBUF], out_hbm.at[pl.ds(base_row + i * CHUNK, CHUNK)],
            sem_s[i % NBUF])

    gather(0).start()
    gather(1).start()
    for i in range(MAX_CPW):
        if i + 2 < MAX_CPW:
            @pl.when(i + 2 < nc)
            def _(i=i):
                if i >= 1:
                    scatter(i - 1).wait()
                gather(i + 2).start()

        @pl.when(i < nc)
        def _(i=i):
            gather(i).wait()
            scatter(i).start()

    scatter(0).wait()
    scatter(1).wait()
    scatter(2).wait()

    @pl.when(wid == NW - 1)
    def _():
        pltpu.async_copy(
            table_hbm.at[idx_v.at[pl.ds(BASE_CPW * CHUNK, TAIL)]],
            rows0.at[pl.ds(0, TAIL)], sg0).wait()
        pltpu.sync_copy(rows0.at[pl.ds(0, TAIL)],
                        out_hbm.at[pl.ds(N_FULL * CHUNK, TAIL)])


@jax.jit
def _run(atomic_numbers, table32):
    mesh = plsc.VectorSubcoreMesh(core_axis_name="c", subcore_axis_name="s")
    f = functools.partial(
        pl.kernel, mesh=mesh,
        out_type=jax.ShapeDtypeStruct((B, D), jnp.float32),
        compiler_params=pltpu.CompilerParams(
            disable_bounds_checks=True,
            disable_semaphore_checks=True,
            skip_device_barrier=True,
        ),
        scratch_types=[
            pltpu.VMEM((IDXBUF,), jnp.int32),
            pltpu.VMEM((CHUNK, D), jnp.float32),
            pltpu.VMEM((CHUNK, D), jnp.float32),
            pltpu.VMEM((CHUNK, D), jnp.float32),
            pltpu.SemaphoreType.DMA,
            pltpu.SemaphoreType.DMA,
            pltpu.SemaphoreType.DMA,
            pltpu.SemaphoreType.DMA,
            pltpu.SemaphoreType.DMA,
            pltpu.SemaphoreType.DMA,
        ],
    )(_gather_kernel)
    return f(atomic_numbers, table32)


def kernel(atomic_numbers, table):
    table32 = jnp.repeat(table, NW, axis=0)
    return _run(atomic_numbers.astype(jnp.int32), table32)
